# baseline (speedup 1.0000x reference)
import functools

import jax
import jax.numpy as jnp
from jax import lax
from jax.experimental import pallas as pl
from jax.experimental.pallas import tpu as pltpu
from jax.experimental.pallas import tpu_sc as plsc

KNB = 20


def _knn_body(base, x_ref, wbt_ref, idx_ref, t2_ref):
    X = x_ref[0]
    N, C = X.shape
    XX = X * X
    sqr = lax.dot_general(
        jnp.ones((1, C), jnp.float32), XX, (((1,), (1,)), ((), ())),
        precision=lax.Precision.HIGHEST)
    G = lax.dot_general(X, X, (((1,), (1,)), ((), ())))
    work = 2.0 * G - sqr
    cols = lax.broadcasted_iota(jnp.int32, (N, N), 1)
    outs = []
    a = None
    for j in range(KNB):
        if j > 0:
            work = jnp.where(cols == a, -jnp.inf, work)
        m = jnp.max(work, axis=1, keepdims=True)
        cand = jnp.where(work == m, cols, N)
        a = jnp.min(cand, axis=1, keepdims=True)
        outs.append(a)
    idxf = jnp.concatenate(outs, axis=1)
    idx_ref[0] = idxf + (base + pl.program_id(0)) * N
    t2_ref[0] = jnp.dot(X, wbt_ref[...], preferred_element_type=jnp.float32)


def _knn_t2(xt, wbt, base):
    B, N, C = xt.shape
    O = wbt.shape[1]
    return pl.pallas_call(
        functools.partial(_knn_body, base),
        grid=(B,),
        in_specs=[pl.BlockSpec((1, N, C), lambda b: (b, 0, 0)),
                  pl.BlockSpec((C, O), lambda b: (0, 0))],
        out_specs=[pl.BlockSpec((1, N, KNB), lambda b: (b, 0, 0)),
                   pl.BlockSpec((1, N, O), lambda b: (b, 0, 0))],
        out_shape=[jax.ShapeDtypeStruct((B, N, KNB), jnp.int32),
                   jax.ShapeDtypeStruct((B, N, O), jnp.float32)],
    )(xt, wbt)


def _conv_body(gx_ref, x_ref, wat_ref, s1_ref, s2_ref, mx_ref, a1, a2, am):
    k = pl.program_id(1)
    d = gx_ref[0, 0] - x_ref[0]
    y = jnp.dot(d, wat_ref[...], preferred_element_type=jnp.float32)

    @pl.when(k == 0)
    def _():
        a1[...] = y
        a2[...] = y * y
        am[...] = y

    @pl.when(k > 0)
    def _():
        a1[...] += y
        a2[...] += y * y
        am[...] = jnp.maximum(am[...], y)

    @pl.when(k == KNB - 1)
    def _():
        s1_ref[0] = a1[...]
        s2_ref[0] = a2[...]
        mx_ref[0] = am[...]


def _conv(gx, xt, wat):
    B, K_, N, C = gx.shape
    O = wat.shape[1]
    outs = [jax.ShapeDtypeStruct((B, N, O), jnp.float32)] * 3
    return pl.pallas_call(
        _conv_body,
        grid=(B, K_),
        in_specs=[pl.BlockSpec((1, 1, N, C), lambda b, k: (b, k, 0, 0)),
                  pl.BlockSpec((1, N, C), lambda b, k: (b, 0, 0)),
                  pl.BlockSpec((C, O), lambda b, k: (0, 0))],
        out_specs=[pl.BlockSpec((1, N, O), lambda b, k: (b, 0, 0))] * 3,
        out_shape=outs,
        scratch_shapes=[pltpu.VMEM((N, O), jnp.float32)] * 3,
    )(gx, xt, wat)


def _stats_body(s1_ref, s2_ref, t2_ref, m_ref, inv_ref, a1, a2):
    b = pl.program_id(0)
    B = pl.num_programs(0)
    s1 = s1_ref[0]
    s2 = s2_ref[0]
    t2 = t2_ref[0]
    p1 = jnp.sum(s1 + KNB * t2, axis=0, keepdims=True)
    p2 = jnp.sum(s2 + 2.0 * t2 * s1 + KNB * (t2 * t2), axis=0, keepdims=True)

    @pl.when(b == 0)
    def _():
        a1[...] = p1
        a2[...] = p2

    @pl.when(b > 0)
    def _():
        a1[...] += p1
        a2[...] += p2

    @pl.when(b == B - 1)
    def _():
        cnt = B * s1.shape[0] * KNB
        m = a1[...] / cnt
        var = a2[...] / cnt - m * m
        m_ref[...] = m
        inv_ref[...] = 1.0 / jnp.sqrt(var + 1e-5)


def _stats(s1, s2, t2):
    B, N, O = s1.shape
    return pl.pallas_call(
        _stats_body,
        grid=(B,),
        in_specs=[pl.BlockSpec((1, N, O), lambda b: (b, 0, 0))] * 3,
        out_specs=[pl.BlockSpec((1, O), lambda b: (0, 0))] * 2,
        out_shape=[jax.ShapeDtypeStruct((1, O), jnp.float32)] * 2,
        scratch_shapes=[pltpu.VMEM((1, O), jnp.float32)] * 2,
    )(s1, s2, t2)


def _norm_body(mx_ref, t2_ref, m_ref, inv_ref, g_ref, bb_ref, o_ref):
    y = (mx_ref[0] + t2_ref[0] - m_ref[...]) * (inv_ref[...] * g_ref[...]) \
        + bb_ref[...]
    o_ref[0] = jnp.where(y > 0, y, 0.2 * y)


def _norm(mx, t2, m, inv, g, bb):
    B, N, O = mx.shape
    return pl.pallas_call(
        _norm_body,
        grid=(B,),
        in_specs=[pl.BlockSpec((1, N, O), lambda b: (b, 0, 0))] * 2
        + [pl.BlockSpec((1, O), lambda b: (0, 0))] * 4,
        out_specs=pl.BlockSpec((1, N, O), lambda b: (b, 0, 0)),
        out_shape=jax.ShapeDtypeStruct((B, N, O), jnp.float32),
    )(mx, t2, m, inv, g.reshape(1, O), bb.reshape(1, O))


def _conv5_body(xc_ref, w5t_ref, s1_ref, s2_ref, mx_ref):
    y = jnp.dot(xc_ref[0], w5t_ref[...], preferred_element_type=jnp.float32)
    s1_ref[0] = jnp.sum(y, axis=0, keepdims=True)
    s2_ref[0] = jnp.sum(y * y, axis=0, keepdims=True)
    mx_ref[0] = jnp.max(y, axis=0, keepdims=True)


def _conv5(xc, w5t):
    B, N, C = xc.shape
    O = w5t.shape[1]
    return pl.pallas_call(
        _conv5_body,
        grid=(B,),
        in_specs=[pl.BlockSpec((1, N, C), lambda b: (b, 0, 0)),
                  pl.BlockSpec((C, O), lambda b: (0, 0))],
        out_specs=[pl.BlockSpec((1, 1, O), lambda b: (b, 0, 0))] * 3,
        out_shape=[jax.ShapeDtypeStruct((B, 1, O), jnp.float32)] * 3,
    )(xc, w5t)


def _final_body(s1_ref, s2_ref, mx_ref, g_ref, bb_ref, o_ref):
    B, O = o_ref.shape
    cnt = B * 1024.0
    m = jnp.sum(s1_ref[...], axis=0, keepdims=True) / cnt
    var = jnp.sum(s2_ref[...], axis=0, keepdims=True) / cnt - m * m
    inv = 1.0 / jnp.sqrt(var + 1e-5)
    y = (mx_ref[...] - m) * (inv * g_ref[...]) + bb_ref[...]
    o_ref[...] = jnp.where(y > 0, y, 0.2 * y)


def _final(s1, s2, mx, g, bb):
    B, O = s1.shape
    return pl.pallas_call(
        _final_body,
        in_specs=[pl.BlockSpec((B, O), lambda: (0, 0))] * 3
        + [pl.BlockSpec((1, O), lambda: (0, 0))] * 2,
        out_specs=pl.BlockSpec((B, O), lambda: (0, 0)),
        out_shape=jax.ShapeDtypeStruct((B, O), jnp.float32),
    )(s1, s2, mx, g.reshape(1, O), bb.reshape(1, O))


_SC_RING = 4


def _sc_gather(xt, idx_t):
    B, _, N = idx_t.shape
    C = xt.shape[2]
    xf = xt.reshape(-1, C)
    info = plsc.get_sparse_core_info()
    NW = info.num_cores * info.num_subcores
    CH = 128
    chunks = (B * N) // CH
    per_w = chunks // NW
    mesh = plsc.VectorSubcoreMesh(core_axis_name="c", subcore_axis_name="s")

    @functools.partial(
        pl.kernel, mesh=mesh,
        out_type=jax.ShapeDtypeStruct((B, KNB, N, C), jnp.float32),
        scratch_types=[pltpu.VMEM((KNB, CH), jnp.int32),
                       pltpu.VMEM((_SC_RING, CH, C), jnp.float32)]
        + [pltpu.SemaphoreType.DMA] * (2 * _SC_RING + 1),
    )
    def k(xf_hbm, idx_hbm, out_hbm, idxblk, gbuf, *sems):
        gsem = sems[:_SC_RING]
        ssem = sems[_SC_RING:2 * _SC_RING]
        isem = sems[2 * _SC_RING]
        wid = lax.axis_index("s") * info.num_cores + lax.axis_index("c")

        cpb = N // CH
        cpb_shift = cpb.bit_length() - 1

        def chunk_body(t, carry):
            ci = wid * per_w + t
            b = ci >> cpb_shift
            n0 = (ci & (cpb - 1)) * CH
            pltpu.async_copy(
                idx_hbm.at[b, :, pl.ds(n0, CH)], idxblk, isem).wait()

            def fire(kk):
                return pltpu.async_copy(
                    xf_hbm.at[idxblk.at[kk]], gbuf.at[kk % _SC_RING],
                    gsem[kk % _SC_RING])

            def store(kk):
                return pltpu.async_copy(
                    gbuf.at[kk % _SC_RING],
                    out_hbm.at[b, kk, pl.ds(n0, CH), :],
                    ssem[kk % _SC_RING])

            gcp = [None] * KNB
            scp = [None] * KNB
            lag = _SC_RING - 1
            for kk in range(KNB):
                if kk >= _SC_RING:
                    scp[kk - _SC_RING].wait()
                gcp[kk] = fire(kk)
                if kk >= lag:
                    j = kk - lag
                    gcp[j].wait()
                    scp[j] = store(j)
            for j in range(KNB - lag, KNB):
                gcp[j].wait()
                scp[j] = store(j)
            for j in range(KNB - _SC_RING, KNB):
                scp[j].wait()
            return carry

        lax.fori_loop(0, per_w, chunk_body, 0)

    return k(xf, idx_t)


def _edge_layer(xt, W, g, b, Cin):
    Cp = xt.shape[2]
    O = W.shape[0]
    wa = jnp.zeros((Cp, O), jnp.float32).at[:Cin].set(W[:, :Cin].T)
    wb = jnp.zeros((Cp, O), jnp.float32).at[:Cin].set(W[:, Cin:].T)
    B = xt.shape[0]
    GB = B // 2
    knns = []
    for gi in range(2):
        sl = slice(gi * GB, (gi + 1) * GB)
        idx, t2g = _knn_t2(xt[sl], wb, gi * GB)
        knns.append((jnp.transpose(idx, (0, 2, 1)), t2g))
    parts = []
    for gi in range(2):
        sl = slice(gi * GB, (gi + 1) * GB)
        gx = _sc_gather(xt, knns[gi][0])
        parts.append(_conv(gx, xt[sl], wa))
    t2 = jnp.concatenate([k[1] for k in knns], axis=0)
    s1, s2, mx = (jnp.concatenate([p[i] for p in parts], axis=0)
                  for i in range(3))
    m, inv = _stats(s1, s2, t2)
    return _norm(mx, t2, m, inv, g, b)


def kernel(x, W1, g1, b1, W2, g2, b2, W3, g3, b3, W4, g4, b4, W5, g5, b5):
    B, N, _ = x.shape

    def pad128(a):
        return jnp.pad(a, ((0, 0), (0, 0), (0, 128 - a.shape[2])))

    x1 = _edge_layer(pad128(x), W1, g1, b1, 3)
    x2 = _edge_layer(pad128(x1), W2, g2, b2, 64)
    x3 = _edge_layer(pad128(x2), W3, g3, b3, 64)
    x4 = _edge_layer(x3, W4, g4, b4, 128)
    xc = jnp.concatenate([x1, x2, x3, x4], axis=2)
    s1, s2, mx = _conv5(xc, W5.T)
    out = _final(s1.reshape(B, -1), s2.reshape(B, -1), mx.reshape(B, -1), g5, b5)
    return out.reshape(B, 1, -1)


# speedup vs baseline: 1.1199x; 1.1199x over previous
import functools

import jax
import jax.numpy as jnp
from jax import lax
from jax.experimental import pallas as pl
from jax.experimental.pallas import tpu as pltpu
from jax.experimental.pallas import tpu_sc as plsc

KNB = 20


def _knn_body(base, x_ref, wbt_ref, idx_ref, t2_ref):
    X = x_ref[0]
    N, C = X.shape
    XX = X * X
    sqr = lax.dot_general(
        jnp.ones((1, C), jnp.float32), XX, (((1,), (1,)), ((), ())),
        precision=lax.Precision.HIGHEST)
    G = lax.dot_general(X, X, (((1,), (1,)), ((), ())))
    work = 2.0 * G - sqr
    cols = lax.broadcasted_iota(jnp.int32, (N, N), 1).astype(jnp.float32)
    outs = []
    a = None
    for j in range(KNB):
        if j > 0:
            work = jnp.where(cols == a, -jnp.inf, work)
        m = jnp.max(work, axis=1, keepdims=True)
        cand = jnp.where(work == m, cols, jnp.float32(N))
        a = jnp.min(cand, axis=1, keepdims=True)
        outs.append(a)
    idxf = jnp.concatenate(outs, axis=1)
    idx_ref[0] = idxf + jnp.float32((base + pl.program_id(0)) * N)
    t2_ref[0] = jnp.dot(X, wbt_ref[...], preferred_element_type=jnp.float32)


def _knn_t2(xt, wbt, base):
    B, N, C = xt.shape
    O = wbt.shape[1]
    return pl.pallas_call(
        functools.partial(_knn_body, base),
        grid=(B,),
        in_specs=[pl.BlockSpec((1, N, C), lambda b: (b, 0, 0)),
                  pl.BlockSpec((C, O), lambda b: (0, 0))],
        out_specs=[pl.BlockSpec((1, N, KNB), lambda b: (b, 0, 0)),
                   pl.BlockSpec((1, N, O), lambda b: (b, 0, 0))],
        out_shape=[jax.ShapeDtypeStruct((B, N, KNB), jnp.float32),
                   jax.ShapeDtypeStruct((B, N, O), jnp.float32)],
    )(xt, wbt)


def _conv_body(gx_ref, x_ref, wat_ref, s1_ref, s2_ref, mx_ref, a1, a2, am):
    k = pl.program_id(1)
    d = gx_ref[0, 0] - x_ref[0]
    y = jnp.dot(d, wat_ref[...], preferred_element_type=jnp.float32)

    @pl.when(k == 0)
    def _():
        a1[...] = y
        a2[...] = y * y
        am[...] = y

    @pl.when(k > 0)
    def _():
        a1[...] += y
        a2[...] += y * y
        am[...] = jnp.maximum(am[...], y)

    @pl.when(k == KNB - 1)
    def _():
        s1_ref[0] = a1[...]
        s2_ref[0] = a2[...]
        mx_ref[0] = am[...]


def _conv(gx, xt, wat):
    B, K_, N, C = gx.shape
    O = wat.shape[1]
    outs = [jax.ShapeDtypeStruct((B, N, O), jnp.float32)] * 3
    return pl.pallas_call(
        _conv_body,
        grid=(B, K_),
        in_specs=[pl.BlockSpec((1, 1, N, C), lambda b, k: (b, k, 0, 0)),
                  pl.BlockSpec((1, N, C), lambda b, k: (b, 0, 0)),
                  pl.BlockSpec((C, O), lambda b, k: (0, 0))],
        out_specs=[pl.BlockSpec((1, N, O), lambda b, k: (b, 0, 0))] * 3,
        out_shape=outs,
        scratch_shapes=[pltpu.VMEM((N, O), jnp.float32)] * 3,
    )(gx, xt, wat)


def _stats_body(s1_ref, s2_ref, t2_ref, m_ref, inv_ref, a1, a2):
    b = pl.program_id(0)
    B = pl.num_programs(0)
    s1 = s1_ref[0]
    s2 = s2_ref[0]
    t2 = t2_ref[0]
    p1 = jnp.sum(s1 + KNB * t2, axis=0, keepdims=True)
    p2 = jnp.sum(s2 + 2.0 * t2 * s1 + KNB * (t2 * t2), axis=0, keepdims=True)

    @pl.when(b == 0)
    def _():
        a1[...] = p1
        a2[...] = p2

    @pl.when(b > 0)
    def _():
        a1[...] += p1
        a2[...] += p2

    @pl.when(b == B - 1)
    def _():
        cnt = B * s1.shape[0] * KNB
        m = a1[...] / cnt
        var = a2[...] / cnt - m * m
        m_ref[...] = m
        inv_ref[...] = 1.0 / jnp.sqrt(var + 1e-5)


def _stats(s1, s2, t2):
    B, N, O = s1.shape
    return pl.pallas_call(
        _stats_body,
        grid=(B,),
        in_specs=[pl.BlockSpec((1, N, O), lambda b: (b, 0, 0))] * 3,
        out_specs=[pl.BlockSpec((1, O), lambda b: (0, 0))] * 2,
        out_shape=[jax.ShapeDtypeStruct((1, O), jnp.float32)] * 2,
        scratch_shapes=[pltpu.VMEM((1, O), jnp.float32)] * 2,
    )(s1, s2, t2)


def _norm_body(mx_ref, t2_ref, m_ref, inv_ref, g_ref, bb_ref, o_ref):
    y = (mx_ref[0] + t2_ref[0] - m_ref[...]) * (inv_ref[...] * g_ref[...]) \
        + bb_ref[...]
    o_ref[0] = jnp.where(y > 0, y, 0.2 * y)


def _norm(mx, t2, m, inv, g, bb):
    B, N, O = mx.shape
    return pl.pallas_call(
        _norm_body,
        grid=(B,),
        in_specs=[pl.BlockSpec((1, N, O), lambda b: (b, 0, 0))] * 2
        + [pl.BlockSpec((1, O), lambda b: (0, 0))] * 4,
        out_specs=pl.BlockSpec((1, N, O), lambda b: (b, 0, 0)),
        out_shape=jax.ShapeDtypeStruct((B, N, O), jnp.float32),
    )(mx, t2, m, inv, g.reshape(1, O), bb.reshape(1, O))


def _conv5_body(xc_ref, w5t_ref, s1_ref, s2_ref, mx_ref):
    y = jnp.dot(xc_ref[0], w5t_ref[...], preferred_element_type=jnp.float32)
    s1_ref[0] = jnp.sum(y, axis=0, keepdims=True)
    s2_ref[0] = jnp.sum(y * y, axis=0, keepdims=True)
    mx_ref[0] = jnp.max(y, axis=0, keepdims=True)


def _conv5(xc, w5t):
    B, N, C = xc.shape
    O = w5t.shape[1]
    return pl.pallas_call(
        _conv5_body,
        grid=(B,),
        in_specs=[pl.BlockSpec((1, N, C), lambda b: (b, 0, 0)),
                  pl.BlockSpec((C, O), lambda b: (0, 0))],
        out_specs=[pl.BlockSpec((1, 1, O), lambda b: (b, 0, 0))] * 3,
        out_shape=[jax.ShapeDtypeStruct((B, 1, O), jnp.float32)] * 3,
    )(xc, w5t)


def _final_body(s1_ref, s2_ref, mx_ref, g_ref, bb_ref, o_ref):
    B, O = o_ref.shape
    cnt = B * 1024.0
    m = jnp.sum(s1_ref[...], axis=0, keepdims=True) / cnt
    var = jnp.sum(s2_ref[...], axis=0, keepdims=True) / cnt - m * m
    inv = 1.0 / jnp.sqrt(var + 1e-5)
    y = (mx_ref[...] - m) * (inv * g_ref[...]) + bb_ref[...]
    o_ref[...] = jnp.where(y > 0, y, 0.2 * y)


def _final(s1, s2, mx, g, bb):
    B, O = s1.shape
    return pl.pallas_call(
        _final_body,
        in_specs=[pl.BlockSpec((B, O), lambda: (0, 0))] * 3
        + [pl.BlockSpec((1, O), lambda: (0, 0))] * 2,
        out_specs=pl.BlockSpec((B, O), lambda: (0, 0)),
        out_shape=jax.ShapeDtypeStruct((B, O), jnp.float32),
    )(s1, s2, mx, g.reshape(1, O), bb.reshape(1, O))


_SC_RING = 4


def _sc_gather(xt, idx_t):
    B, _, N = idx_t.shape
    C = xt.shape[2]
    xf = xt.reshape(-1, C)
    info = plsc.get_sparse_core_info()
    NW = info.num_cores * info.num_subcores
    CH = 128
    chunks = (B * N) // CH
    per_w = chunks // NW
    mesh = plsc.VectorSubcoreMesh(core_axis_name="c", subcore_axis_name="s")

    @functools.partial(
        pl.kernel, mesh=mesh,
        out_type=jax.ShapeDtypeStruct((B, KNB, N, C), jnp.float32),
        scratch_types=[pltpu.VMEM((KNB, CH), jnp.int32),
                       pltpu.VMEM((_SC_RING, CH, C), jnp.float32)]
        + [pltpu.SemaphoreType.DMA] * (2 * _SC_RING + 1),
    )
    def k(xf_hbm, idx_hbm, out_hbm, idxblk, gbuf, *sems):
        gsem = sems[:_SC_RING]
        ssem = sems[_SC_RING:2 * _SC_RING]
        isem = sems[2 * _SC_RING]
        wid = lax.axis_index("s") * info.num_cores + lax.axis_index("c")

        cpb = N // CH
        cpb_shift = cpb.bit_length() - 1

        def chunk_body(t, carry):
            ci = wid * per_w + t
            b = ci >> cpb_shift
            n0 = (ci & (cpb - 1)) * CH
            pltpu.async_copy(
                idx_hbm.at[b, :, pl.ds(n0, CH)], idxblk, isem).wait()

            def fire(kk):
                return pltpu.async_copy(
                    xf_hbm.at[idxblk.at[kk]], gbuf.at[kk % _SC_RING],
                    gsem[kk % _SC_RING])

            def store(kk):
                return pltpu.async_copy(
                    gbuf.at[kk % _SC_RING],
                    out_hbm.at[b, kk, pl.ds(n0, CH), :],
                    ssem[kk % _SC_RING])

            gcp = [None] * KNB
            scp = [None] * KNB
            lag = _SC_RING - 1
            for kk in range(KNB):
                if kk >= _SC_RING:
                    scp[kk - _SC_RING].wait()
                gcp[kk] = fire(kk)
                if kk >= lag:
                    j = kk - lag
                    gcp[j].wait()
                    scp[j] = store(j)
            for j in range(KNB - lag, KNB):
                gcp[j].wait()
                scp[j] = store(j)
            for j in range(KNB - _SC_RING, KNB):
                scp[j].wait()
            return carry

        lax.fori_loop(0, per_w, chunk_body, 0)

    return k(xf, idx_t)


def _edge_layer(xt, W, g, b, Cin):
    Cp = xt.shape[2]
    O = W.shape[0]
    wa = jnp.zeros((Cp, O), jnp.float32).at[:Cin].set(W[:, :Cin].T)
    wb = jnp.zeros((Cp, O), jnp.float32).at[:Cin].set(W[:, Cin:].T)
    B = xt.shape[0]
    GB = B // 2
    knns = []
    for gi in range(2):
        sl = slice(gi * GB, (gi + 1) * GB)
        idx, t2g = _knn_t2(xt[sl], wb, gi * GB)
        idx_t = jnp.transpose(idx, (0, 2, 1)).astype(jnp.int32)
        knns.append((idx_t, t2g))
    parts = []
    for gi in range(2):
        sl = slice(gi * GB, (gi + 1) * GB)
        gx = _sc_gather(xt, knns[gi][0])
        parts.append(_conv(gx, xt[sl], wa))
    t2 = jnp.concatenate([k[1] for k in knns], axis=0)
    s1, s2, mx = (jnp.concatenate([p[i] for p in parts], axis=0)
                  for i in range(3))
    m, inv = _stats(s1, s2, t2)
    return _norm(mx, t2, m, inv, g, b)


def kernel(x, W1, g1, b1, W2, g2, b2, W3, g3, b3, W4, g4, b4, W5, g5, b5):
    B, N, _ = x.shape

    def pad128(a):
        return jnp.pad(a, ((0, 0), (0, 0), (0, 128 - a.shape[2])))

    x1 = _edge_layer(pad128(x), W1, g1, b1, 3)
    x2 = _edge_layer(pad128(x1), W2, g2, b2, 64)
    x3 = _edge_layer(pad128(x2), W3, g3, b3, 64)
    x4 = _edge_layer(x3, W4, g4, b4, 128)
    xc = jnp.concatenate([x1, x2, x3, x4], axis=2)
    s1, s2, mx = _conv5(xc, W5.T)
    out = _final(s1.reshape(B, -1), s2.reshape(B, -1), mx.reshape(B, -1), g5, b5)
    return out.reshape(B, 1, -1)
